# Initial kernel scaffold; baseline (speedup 1.0000x reference)
import jax
import jax.numpy as jnp
from jax.experimental import pallas as pl


def kernel(x, edge_index, Wz, bz, Wr, br, Wh, bh, Wlz, blz, Wlr, blr, Wlh, blh, Wout, bout):
    raise NotImplementedError("write your pallas kernel here")


# speedup vs baseline: 41.5475x; 41.5475x over previous
import functools

import jax
import jax.numpy as jnp
from jax import lax
from jax.experimental import pallas as pl
from jax.experimental.pallas import tpu as pltpu
from jax.experimental.pallas import tpu_sc as plsc

N = 100000
E = 1600000
T = 4
F = 32
HID = 64
OUT = 16

NC = 2
NS = 16
CHUNK = 128
KCH = 8
NBLK = 100
CPT = NBLK * KCH
EPT = CPT * CHUNK
EP = NS * EPT
NROW = 100352
RPT = NROW // NS

BN2 = 3136
BM = 2000

_f32 = jnp.float32


def _deg_body(didx, z1, ones_h, deg_out, ones_v, idx_v, acc0, acc1, ssem):
    cid = lax.axis_index("c")
    sid = lax.axis_index("s")
    pltpu.sync_copy(ones_h, ones_v)
    for tt, acc in enumerate((acc0, acc1)):
        t = cid * (T // NC) + tt
        pltpu.sync_copy(z1.at[pl.ds(sid * RPT, RPT)],
                        acc.at[pl.ds(sid * RPT, RPT)])
        plsc.subcore_barrier()

        def body(b, carry, acc=acc, t=t):
            pltpu.sync_copy(didx.at[t, sid, pl.ds(b * KCH, KCH)], idx_v)
            descs = [
                pltpu.async_copy(ones_v, acc.at[idx_v.at[j]], ssem, add=True)
                for j in range(KCH)
            ]
            for d in descs:
                d.wait()
            return carry

        lax.fori_loop(0, NBLK, body, 0)
        plsc.subcore_barrier()
        pltpu.sync_copy(acc.at[pl.ds(sid * RPT, RPT)],
                        deg_out.at[t, pl.ds(sid * RPT, RPT)])
        plsc.subcore_barrier()


def _deg_call(dst_p, z1, ones128):
    mesh = plsc.VectorSubcoreMesh(core_axis_name="c", subcore_axis_name="s")
    kern = pl.kernel(
        _deg_body,
        out_type=jax.ShapeDtypeStruct((T, NROW), _f32),
        mesh=mesh,
        compiler_params=pltpu.CompilerParams(use_tc_tiling_on_sc=False),
        scratch_types=[
            pltpu.VMEM((CHUNK,), _f32),
            pltpu.VMEM((KCH, CHUNK), jnp.int32),
            pltpu.VMEM_SHARED((NROW,), _f32),
            pltpu.VMEM_SHARED((NROW,), _f32),
            pltpu.SemaphoreType.DMA,
        ],
    )
    return kern(dst_p, z1, ones128)


def _xn_body(x_ref, deg_ref, out_ref):
    i = pl.program_id(2)
    di = lax.rsqrt(deg_ref[0] + 1.0)
    xn = x_ref[0, 0] * di
    glob = lax.broadcasted_iota(jnp.int32, (BN2, 1), 0) + i * BN2
    out_ref[0, 0] = jnp.where(glob < N, xn, 0.0)


def _xn_call(x_s, deg_r):
    return pl.pallas_call(
        _xn_body,
        grid=(NC, T, NROW // BN2),
        in_specs=[
            pl.BlockSpec((1, 1, BN2, 16), lambda c, t, i: (c, t, i, 0)),
            pl.BlockSpec((1, BN2, 1), lambda c, t, i: (t, i, 0)),
        ],
        out_specs=pl.BlockSpec((1, 1, BN2, 16), lambda c, t, i: (c, t, i, 0)),
        out_shape=jax.ShapeDtypeStruct((NC, T, NROW, 16), _f32),
    )(x_s, deg_r)


def _agg_body(sidx, didx, tbl, z16, agg_out,
              sidx_v, didx_v, rows, acc, gsem, ssem):
    cid = lax.axis_index("c")
    sid = lax.axis_index("s")
    for t in range(T):
        pltpu.sync_copy(z16.at[pl.ds(sid * RPT, RPT)],
                        acc.at[pl.ds(sid * RPT, RPT)])
        plsc.subcore_barrier()
        tbl_t = tbl.at[cid, t]

        def body(b, carry, tbl_t=tbl_t, t=t):
            pltpu.sync_copy(sidx.at[t, sid, pl.ds(b * KCH, KCH)], sidx_v)
            pltpu.sync_copy(didx.at[t, sid, pl.ds(b * KCH, KCH)], didx_v)
            g = [
                pltpu.async_copy(tbl_t.at[sidx_v.at[j]], rows.at[j], gsem)
                for j in range(KCH)
            ]
            for d in g:
                d.wait()
            s = [
                pltpu.async_copy(rows.at[j], acc.at[didx_v.at[j]], ssem,
                                 add=True)
                for j in range(KCH)
            ]
            for d in s:
                d.wait()
            return carry

        lax.fori_loop(0, NBLK, body, 0)
        plsc.subcore_barrier()
        pltpu.sync_copy(acc.at[pl.ds(sid * RPT, RPT)],
                        agg_out.at[cid, t, pl.ds(sid * RPT, RPT)])
        plsc.subcore_barrier()


def _agg_call(src_p, dst_p, tbl, z16):
    mesh = plsc.VectorSubcoreMesh(core_axis_name="c", subcore_axis_name="s")
    kern = pl.kernel(
        _agg_body,
        out_type=jax.ShapeDtypeStruct((NC, T, NROW, 16), _f32),
        mesh=mesh,
        compiler_params=pltpu.CompilerParams(use_tc_tiling_on_sc=False),
        scratch_types=[
            pltpu.VMEM((KCH, CHUNK), jnp.int32),
            pltpu.VMEM((KCH, CHUNK), jnp.int32),
            pltpu.VMEM((KCH, CHUNK, 16), _f32),
            pltpu.VMEM_SHARED((NROW, 16), _f32),
            pltpu.SemaphoreType.DMA,
            pltpu.SemaphoreType.DMA,
        ],
    )
    return kern(src_p, dst_p, tbl, z16)


def _main_body(xs_ref, agg_ref, deg_ref,
               Wz_ref, Wr_ref, Wh_ref, Wlz_ref, Wlr_ref, Wlh_ref,
               bz_ref, br_ref, bh_ref, blz_ref, blr_ref, blh_ref,
               Wout_ref, bout_ref, out_ref):
    dot = functools.partial(jnp.dot, preferred_element_type=_f32)
    Wlz = Wlz_ref[...]
    Wlr = Wlr_ref[...]
    Wlh = Wlh_ref[...]
    Az = dot(Wz_ref[...], Wlz[:HID])
    Ar = dot(Wr_ref[...], Wlr[:HID])
    Ah = dot(Wh_ref[...], Wlh[:HID])
    Bz, Br, Bh = Wlz[HID:], Wlr[HID:], Wlh[HID:]
    cz = dot(bz_ref[...], Wlz[:HID]) + blz_ref[...]
    cr = dot(br_ref[...], Wlr[:HID]) + blr_ref[...]
    ch = dot(bh_ref[...], Wlh[:HID]) + blh_ref[...]

    H = jnp.zeros((BM, HID), _f32)
    for t in range(T):
        di = lax.rsqrt(deg_ref[t] + 1.0)
        x0, x1 = xs_ref[0, t], xs_ref[1, t]
        a0, a1 = agg_ref[0, t], agg_ref[1, t]
        xa0 = (a0 + x0 * di) * di
        xa1 = (a1 + x1 * di) * di
        Z = jax.nn.sigmoid(dot(xa0, Az[:16]) + dot(xa1, Az[16:])
                           + dot(H, Bz) + cz)
        R = jax.nn.sigmoid(dot(xa0, Ar[:16]) + dot(xa1, Ar[16:])
                           + dot(H, Br) + cr)
        Ht = jnp.tanh(dot(xa0, Ah[:16]) + dot(xa1, Ah[16:])
                      + dot(H * R, Bh) + ch)
        H = Z * H + (1.0 - Z) * Ht
    out_ref[...] = dot(H, Wout_ref[...]) + bout_ref[...]


def _main_call(x_s, agg, deg_r, Wz, Wr, Wh, Wlz, Wlr, Wlh,
               bz, br, bh, blz, blr, blh, Wout, bout):
    full = lambda shape: pl.BlockSpec(shape, lambda i: tuple(0 for _ in shape))
    return pl.pallas_call(
        _main_body,
        grid=(N // BM,),
        in_specs=[
            pl.BlockSpec((NC, T, BM, 16), lambda i: (0, 0, i, 0)),
            pl.BlockSpec((NC, T, BM, 16), lambda i: (0, 0, i, 0)),
            pl.BlockSpec((T, BM, 1), lambda i: (0, i, 0)),
            full((F, HID)), full((F, HID)), full((F, HID)),
            full((2 * HID, HID)), full((2 * HID, HID)), full((2 * HID, HID)),
            full((1, HID)), full((1, HID)), full((1, HID)),
            full((1, HID)), full((1, HID)), full((1, HID)),
            full((HID, OUT)), full((1, OUT)),
        ],
        out_specs=pl.BlockSpec((BM, OUT), lambda i: (i, 0)),
        out_shape=jax.ShapeDtypeStruct((N, OUT), _f32),
    )(x_s, agg, deg_r, Wz, Wr, Wh, Wlz, Wlr, Wlh,
      bz, br, bh, blz, blr, blh, Wout, bout)


def kernel(x, edge_index, Wz, bz, Wr, br, Wh, bh,
           Wlz, blz, Wlr, blr, Wlh, blh, Wout, bout):
    src = edge_index[:, 0, :]
    dst = edge_index[:, 1, :]
    pad = EP - E
    src_p = jnp.pad(src, ((0, 0), (0, pad)), constant_values=N)
    dst_p = jnp.pad(dst, ((0, 0), (0, pad)), constant_values=N)
    src_p = src_p.reshape(T, NS, CPT, CHUNK)
    dst_p = dst_p.reshape(T, NS, CPT, CHUNK)
    x_s = jnp.stack([x[:, :, :16], x[:, :, 16:]])

    z1 = jnp.zeros((NROW,), _f32)
    z16 = jnp.zeros((NROW, 16), _f32)
    ones128 = jnp.ones((CHUNK,), _f32)

    deg = _deg_call(dst_p, z1, ones128)
    deg_r = deg.reshape(T, NROW, 1)
    tbl = _xn_call(x_s, deg_r)
    agg = _agg_call(src_p, dst_p, tbl, z16)
    return _main_call(
        x_s, agg, deg_r, Wz, Wr, Wh, Wlz, Wlr, Wlh,
        bz.reshape(1, HID), br.reshape(1, HID), bh.reshape(1, HID),
        blz.reshape(1, HID), blr.reshape(1, HID), blh.reshape(1, HID),
        Wout, bout.reshape(1, OUT))


# speedup vs baseline: 44.1205x; 1.0619x over previous
import functools

import jax
import jax.numpy as jnp
from jax import lax
from jax.experimental import pallas as pl
from jax.experimental.pallas import tpu as pltpu
from jax.experimental.pallas import tpu_sc as plsc

N = 100000
E = 1600000
T = 4
F = 32
HID = 64
OUT = 16

NC = 2
NS = 16
CHUNK = 128
E2 = E + N
KCH = 5
NBLK = 168
PAIRS = NBLK // 2
KCH_D = 8
NBLK_D = 105
CPT = NBLK * KCH
EPT = CPT * CHUNK
EP = NS * EPT
NROW = 100352
RPT = NROW // NS

BN2 = 3136
BM = 2000

_f32 = jnp.float32


def _deg_body(didx, z1, ones_h, deg_out, ones_v, idx_v, acc0, acc1, ssem):
    cid = lax.axis_index("c")
    sid = lax.axis_index("s")
    pltpu.sync_copy(ones_h, ones_v)
    for tt, acc in enumerate((acc0, acc1)):
        t = cid * (T // NC) + tt
        pltpu.sync_copy(z1.at[pl.ds(sid * RPT, RPT)],
                        acc.at[pl.ds(sid * RPT, RPT)])
        plsc.subcore_barrier()

        def body(b, carry, acc=acc, t=t):
            pltpu.sync_copy(didx.at[t, sid, pl.ds(b * KCH_D, KCH_D)], idx_v)
            descs = [
                pltpu.async_copy(ones_v, acc.at[idx_v.at[j]], ssem, add=True)
                for j in range(KCH_D)
            ]
            for d in descs:
                d.wait()
            return carry

        lax.fori_loop(0, NBLK_D, body, 0)
        plsc.subcore_barrier()
        pltpu.sync_copy(acc.at[pl.ds(sid * RPT, RPT)],
                        deg_out.at[t, pl.ds(sid * RPT, RPT)])
        plsc.subcore_barrier()


def _deg_call(dst_p, z1, ones128):
    mesh = plsc.VectorSubcoreMesh(core_axis_name="c", subcore_axis_name="s")
    kern = pl.kernel(
        _deg_body,
        out_type=jax.ShapeDtypeStruct((T, NROW), _f32),
        mesh=mesh,
        compiler_params=pltpu.CompilerParams(use_tc_tiling_on_sc=False),
        scratch_types=[
            pltpu.VMEM((CHUNK,), _f32),
            pltpu.VMEM((KCH_D, CHUNK), jnp.int32),
            pltpu.VMEM_SHARED((NROW,), _f32),
            pltpu.VMEM_SHARED((NROW,), _f32),
            pltpu.SemaphoreType.DMA,
        ],
    )
    return kern(dst_p, z1, ones128)


def _xn_body(x_ref, deg_ref, out_ref):
    i = pl.program_id(2)
    di = lax.rsqrt(deg_ref[0])
    xn = x_ref[0, 0] * di
    glob = lax.broadcasted_iota(jnp.int32, (BN2, 1), 0) + i * BN2
    out_ref[0, 0] = jnp.where(glob < N, xn, 0.0)


def _xn_call(x_s, deg_r):
    return pl.pallas_call(
        _xn_body,
        grid=(NC, T, NROW // BN2),
        in_specs=[
            pl.BlockSpec((1, 1, BN2, 16), lambda c, t, i: (c, t, i, 0)),
            pl.BlockSpec((1, BN2, 1), lambda c, t, i: (t, i, 0)),
        ],
        out_specs=pl.BlockSpec((1, 1, BN2, 16), lambda c, t, i: (c, t, i, 0)),
        out_shape=jax.ShapeDtypeStruct((NC, T, NROW, 16), _f32),
    )(x_s, deg_r)


def _agg_body(sidx, didx, tbl, z16, agg_out,
              sidx_v, didx_v, rows, acc, gsems, ssems):
    cid = lax.axis_index("c")
    sid = lax.axis_index("s")
    for t in range(T):
        pltpu.sync_copy(z16.at[pl.ds(sid * RPT, RPT)],
                        acc.at[pl.ds(sid * RPT, RPT)])
        plsc.subcore_barrier()
        tbl_t = tbl.at[cid, t]

        def fire_g(b, p, tbl_t=tbl_t, t=t):
            pltpu.sync_copy(sidx.at[t, sid, pl.ds(b * KCH, KCH)],
                            sidx_v.at[p])
            pltpu.sync_copy(didx.at[t, sid, pl.ds(b * KCH, KCH)],
                            didx_v.at[p])
            for j in range(KCH):
                pltpu.async_copy(tbl_t.at[sidx_v.at[p, j]], rows.at[p, j],
                                 gsems.at[p])

        def wait_g(p, tbl_t=tbl_t):
            for j in range(KCH):
                pltpu.make_async_copy(tbl_t.at[sidx_v.at[p, j]],
                                      rows.at[p, j], gsems.at[p]).wait()

        def fire_s(p):
            for j in range(KCH):
                pltpu.async_copy(rows.at[p, j], acc.at[didx_v.at[p, j]],
                                 ssems.at[p], add=True)

        def wait_s(p):
            for j in range(KCH):
                pltpu.make_async_copy(rows.at[p, j], acc.at[didx_v.at[p, j]],
                                      ssems.at[p]).wait()

        fire_g(0, 0)

        def body(i, carry):
            b = 2 * i
            wait_g(0)
            fire_g(b + 1, 1)
            fire_s(0)
            wait_g(1)
            wait_s(0)

            @pl.when(i < PAIRS - 1)
            def _():
                fire_g(b + 2, 0)
            fire_s(1)
            wait_s(1)
            return carry

        lax.fori_loop(0, PAIRS, body, 0)
        plsc.subcore_barrier()
        pltpu.sync_copy(acc.at[pl.ds(sid * RPT, RPT)],
                        agg_out.at[cid, t, pl.ds(sid * RPT, RPT)])
        plsc.subcore_barrier()


def _agg_call(src_p, dst_p, tbl, z16):
    mesh = plsc.VectorSubcoreMesh(core_axis_name="c", subcore_axis_name="s")
    kern = pl.kernel(
        _agg_body,
        out_type=jax.ShapeDtypeStruct((NC, T, NROW, 16), _f32),
        mesh=mesh,
        compiler_params=pltpu.CompilerParams(use_tc_tiling_on_sc=False),
        scratch_types=[
            pltpu.VMEM((2, KCH, CHUNK), jnp.int32),
            pltpu.VMEM((2, KCH, CHUNK), jnp.int32),
            pltpu.VMEM((2, KCH, CHUNK, 16), _f32),
            pltpu.VMEM_SHARED((NROW, 16), _f32),
            pltpu.SemaphoreType.DMA((2,)),
            pltpu.SemaphoreType.DMA((2,)),
        ],
    )
    return kern(src_p, dst_p, tbl, z16)


def _main_body(agg_ref, deg_ref,
               Wz_ref, Wr_ref, Wh_ref, Wlz_ref, Wlr_ref, Wlh_ref,
               bz_ref, br_ref, bh_ref, blz_ref, blr_ref, blh_ref,
               Wout_ref, bout_ref, out_ref):
    dot = functools.partial(jnp.dot, preferred_element_type=_f32)
    Wlz = Wlz_ref[...]
    Wlr = Wlr_ref[...]
    Wlh = Wlh_ref[...]
    Az = dot(Wz_ref[...], Wlz[:HID])
    Ar = dot(Wr_ref[...], Wlr[:HID])
    Ah = dot(Wh_ref[...], Wlh[:HID])
    Bz, Br, Bh = Wlz[HID:], Wlr[HID:], Wlh[HID:]
    cz = dot(bz_ref[...], Wlz[:HID]) + blz_ref[...]
    cr = dot(br_ref[...], Wlr[:HID]) + blr_ref[...]
    ch = dot(bh_ref[...], Wlh[:HID]) + blh_ref[...]

    H = jnp.zeros((BM, HID), _f32)
    for t in range(T):
        di = lax.rsqrt(deg_ref[t])
        a0, a1 = agg_ref[0, t], agg_ref[1, t]
        xa0 = a0 * di
        xa1 = a1 * di
        Z = jax.nn.sigmoid(dot(xa0, Az[:16]) + dot(xa1, Az[16:])
                           + dot(H, Bz) + cz)
        R = jax.nn.sigmoid(dot(xa0, Ar[:16]) + dot(xa1, Ar[16:])
                           + dot(H, Br) + cr)
        Ht = jnp.tanh(dot(xa0, Ah[:16]) + dot(xa1, Ah[16:])
                      + dot(H * R, Bh) + ch)
        H = Z * H + (1.0 - Z) * Ht
    out_ref[...] = dot(H, Wout_ref[...]) + bout_ref[...]


def _main_call(agg, deg_r, Wz, Wr, Wh, Wlz, Wlr, Wlh,
               bz, br, bh, blz, blr, blh, Wout, bout):
    full = lambda shape: pl.BlockSpec(shape, lambda i: tuple(0 for _ in shape))
    return pl.pallas_call(
        _main_body,
        grid=(N // BM,),
        in_specs=[
            pl.BlockSpec((NC, T, BM, 16), lambda i: (0, 0, i, 0)),
            pl.BlockSpec((T, BM, 1), lambda i: (0, i, 0)),
            full((F, HID)), full((F, HID)), full((F, HID)),
            full((2 * HID, HID)), full((2 * HID, HID)), full((2 * HID, HID)),
            full((1, HID)), full((1, HID)), full((1, HID)),
            full((1, HID)), full((1, HID)), full((1, HID)),
            full((HID, OUT)), full((1, OUT)),
        ],
        out_specs=pl.BlockSpec((BM, OUT), lambda i: (i, 0)),
        out_shape=jax.ShapeDtypeStruct((N, OUT), _f32),
    )(agg, deg_r, Wz, Wr, Wh, Wlz, Wlr, Wlh,
      bz, br, bh, blz, blr, blh, Wout, bout)


def kernel(x, edge_index, Wz, bz, Wr, br, Wh, bh,
           Wlz, blz, Wlr, blr, Wlh, blh, Wout, bout):
    loop = jnp.broadcast_to(jnp.arange(N, dtype=jnp.int32), (T, 1, N))
    ei = jnp.concatenate([edge_index, jnp.tile(loop, (1, 2, 1))], axis=2)
    src = ei[:, 0, :]
    dst = ei[:, 1, :]
    pad = EP - E2
    src_p = jnp.pad(src, ((0, 0), (0, pad)), constant_values=N)
    dst_p = jnp.pad(dst, ((0, 0), (0, pad)), constant_values=N)
    src_p = src_p.reshape(T, NS, CPT, CHUNK)
    dst_p = dst_p.reshape(T, NS, CPT, CHUNK)
    x_s = jnp.stack([x[:, :, :16], x[:, :, 16:]])

    z1 = jnp.zeros((NROW,), _f32)
    z16 = jnp.zeros((NROW, 16), _f32)
    ones128 = jnp.ones((CHUNK,), _f32)

    deg = _deg_call(dst_p, z1, ones128)
    deg_r = deg.reshape(T, NROW, 1)
    tbl = _xn_call(x_s, deg_r)
    agg = _agg_call(src_p, dst_p, tbl, z16)
    return _main_call(
        agg, deg_r, Wz, Wr, Wh, Wlz, Wlr, Wlh,
        bz.reshape(1, HID), br.reshape(1, HID), bh.reshape(1, HID),
        blz.reshape(1, HID), blr.reshape(1, HID), blh.reshape(1, HID),
        Wout, bout.reshape(1, OUT))


# speedup vs baseline: 49.0322x; 1.1113x over previous
import functools

import jax
import jax.numpy as jnp
from jax import lax
from jax.experimental import pallas as pl
from jax.experimental.pallas import tpu as pltpu
from jax.experimental.pallas import tpu_sc as plsc

N = 100000
E = 1600000
T = 4
F = 32
HID = 64
OUT = 16

NC = 2
NS = 16
CHUNK = 128
E2 = E + N
KCH = 5
NBLK = 168
PAIRS = NBLK // 2
KCH_D = 8
NBLK_D = 105
CPT = NBLK * KCH
EPT = CPT * CHUNK
EP = NS * EPT
NROW = 100352
RPT = NROW // NS

BN2 = 3136
BM = 2000

_f32 = jnp.float32


def _deg_body(didx, z1, ones_h, deg_out, ones_v, idx_v, acc0, acc1, ssem):
    cid = lax.axis_index("c")
    sid = lax.axis_index("s")
    pltpu.sync_copy(ones_h, ones_v)
    for tt, acc in enumerate((acc0, acc1)):
        t = cid * (T // NC) + tt
        pltpu.sync_copy(z1.at[pl.ds(sid * RPT, RPT)],
                        acc.at[pl.ds(sid * RPT, RPT)])
        plsc.subcore_barrier()

        def body(b, carry, acc=acc, t=t):
            pltpu.sync_copy(didx.at[t, sid, pl.ds(b * KCH_D, KCH_D)], idx_v)
            descs = [
                pltpu.async_copy(ones_v, acc.at[idx_v.at[j]], ssem, add=True)
                for j in range(KCH_D)
            ]
            for d in descs:
                d.wait()
            return carry

        lax.fori_loop(0, NBLK_D, body, 0)
        plsc.subcore_barrier()
        pltpu.sync_copy(acc.at[pl.ds(sid * RPT, RPT)],
                        deg_out.at[t, pl.ds(sid * RPT, RPT)])
        plsc.subcore_barrier()


def _deg_call(dst_p, z1, ones128):
    mesh = plsc.VectorSubcoreMesh(core_axis_name="c", subcore_axis_name="s")
    kern = pl.kernel(
        _deg_body,
        out_type=jax.ShapeDtypeStruct((T, NROW), _f32),
        mesh=mesh,
        compiler_params=pltpu.CompilerParams(use_tc_tiling_on_sc=False),
        scratch_types=[
            pltpu.VMEM((CHUNK,), _f32),
            pltpu.VMEM((KCH_D, CHUNK), jnp.int32),
            pltpu.VMEM_SHARED((NROW,), _f32),
            pltpu.VMEM_SHARED((NROW,), _f32),
            pltpu.SemaphoreType.DMA,
        ],
    )
    return kern(dst_p, z1, ones128)


def _xn_body(x_ref, deg_ref, out_ref):
    i = pl.program_id(1)
    di = lax.rsqrt(deg_ref[0])
    xn = x_ref[0, 0] * di
    glob = lax.broadcasted_iota(jnp.int32, (BN2, 1), 0) + i * BN2
    out_ref[0] = jnp.where(glob < N, xn, 0.0)


def _xn_call(x_s, deg_r, t):
    return pl.pallas_call(
        _xn_body,
        grid=(NC, NROW // BN2),
        in_specs=[
            pl.BlockSpec((1, 1, BN2, 16), lambda c, i: (c, t, i, 0)),
            pl.BlockSpec((1, BN2, 1), lambda c, i: (t, i, 0)),
        ],
        out_specs=pl.BlockSpec((1, BN2, 16), lambda c, i: (c, i, 0)),
        out_shape=jax.ShapeDtypeStruct((NC, NROW, 16), _f32),
    )(x_s, deg_r)


def _agg_body(t, sidx, didx, tbl, z16, agg_out,
              sidx_v, didx_v, rows, acc, gsems, ssems):
    cid = lax.axis_index("c")
    sid = lax.axis_index("s")
    if True:
        pltpu.sync_copy(z16.at[pl.ds(sid * RPT, RPT)],
                        acc.at[pl.ds(sid * RPT, RPT)])
        plsc.subcore_barrier()
        tbl_t = tbl.at[cid]

        def fire_g(b, p, tbl_t=tbl_t, t=t):
            pltpu.sync_copy(sidx.at[t, sid, pl.ds(b * KCH, KCH)],
                            sidx_v.at[p])
            pltpu.sync_copy(didx.at[t, sid, pl.ds(b * KCH, KCH)],
                            didx_v.at[p])
            for j in range(KCH):
                pltpu.async_copy(tbl_t.at[sidx_v.at[p, j]], rows.at[p, j],
                                 gsems.at[p])

        def wait_g(p, tbl_t=tbl_t):
            for j in range(KCH):
                pltpu.make_async_copy(tbl_t.at[sidx_v.at[p, j]],
                                      rows.at[p, j], gsems.at[p]).wait()

        def fire_s(p):
            for j in range(KCH):
                pltpu.async_copy(rows.at[p, j], acc.at[didx_v.at[p, j]],
                                 ssems.at[p], add=True)

        def wait_s(p):
            for j in range(KCH):
                pltpu.make_async_copy(rows.at[p, j], acc.at[didx_v.at[p, j]],
                                      ssems.at[p]).wait()

        fire_g(0, 0)

        def body(i, carry):
            b = 2 * i
            wait_g(0)
            fire_g(b + 1, 1)
            fire_s(0)
            wait_g(1)
            wait_s(0)

            @pl.when(i < PAIRS - 1)
            def _():
                fire_g(b + 2, 0)
            fire_s(1)
            wait_s(1)
            return carry

        lax.fori_loop(0, PAIRS, body, 0)
        plsc.subcore_barrier()
        pltpu.sync_copy(acc.at[pl.ds(sid * RPT, RPT)],
                        agg_out.at[cid, pl.ds(sid * RPT, RPT)])
        plsc.subcore_barrier()


def _agg_call(src_p, dst_p, tbl, z16, t):
    mesh = plsc.VectorSubcoreMesh(core_axis_name="c", subcore_axis_name="s")
    kern = pl.kernel(
        functools.partial(_agg_body, t),
        out_type=jax.ShapeDtypeStruct((NC, NROW, 16), _f32),
        mesh=mesh,
        compiler_params=pltpu.CompilerParams(use_tc_tiling_on_sc=False),
        scratch_types=[
            pltpu.VMEM((2, KCH, CHUNK), jnp.int32),
            pltpu.VMEM((2, KCH, CHUNK), jnp.int32),
            pltpu.VMEM((2, KCH, CHUNK, 16), _f32),
            pltpu.VMEM_SHARED((NROW, 16), _f32),
            pltpu.SemaphoreType.DMA((2,)),
            pltpu.SemaphoreType.DMA((2,)),
        ],
    )
    return kern(src_p, dst_p, tbl, z16)


def _main_body(a0_ref, a1_ref, a2_ref, a3_ref, deg_ref,
               Wz_ref, Wr_ref, Wh_ref, Wlz_ref, Wlr_ref, Wlh_ref,
               bz_ref, br_ref, bh_ref, blz_ref, blr_ref, blh_ref,
               Wout_ref, bout_ref, out_ref):
    agg_refs = (a0_ref, a1_ref, a2_ref, a3_ref)
    dot = functools.partial(jnp.dot, preferred_element_type=_f32)
    Wlz = Wlz_ref[...]
    Wlr = Wlr_ref[...]
    Wlh = Wlh_ref[...]
    Az = dot(Wz_ref[...], Wlz[:HID])
    Ar = dot(Wr_ref[...], Wlr[:HID])
    Ah = dot(Wh_ref[...], Wlh[:HID])
    Bz, Br, Bh = Wlz[HID:], Wlr[HID:], Wlh[HID:]
    cz = dot(bz_ref[...], Wlz[:HID]) + blz_ref[...]
    cr = dot(br_ref[...], Wlr[:HID]) + blr_ref[...]
    ch = dot(bh_ref[...], Wlh[:HID]) + blh_ref[...]

    H = jnp.zeros((BM, HID), _f32)
    for t in range(T):
        di = lax.rsqrt(deg_ref[t])
        a0, a1 = agg_refs[t][0], agg_refs[t][1]
        xa0 = a0 * di
        xa1 = a1 * di
        Z = jax.nn.sigmoid(dot(xa0, Az[:16]) + dot(xa1, Az[16:])
                           + dot(H, Bz) + cz)
        R = jax.nn.sigmoid(dot(xa0, Ar[:16]) + dot(xa1, Ar[16:])
                           + dot(H, Br) + cr)
        Ht = jnp.tanh(dot(xa0, Ah[:16]) + dot(xa1, Ah[16:])
                      + dot(H * R, Bh) + ch)
        H = Z * H + (1.0 - Z) * Ht
    out_ref[...] = dot(H, Wout_ref[...]) + bout_ref[...]


def _main_call(aggs, deg_r, Wz, Wr, Wh, Wlz, Wlr, Wlh,
               bz, br, bh, blz, blr, blh, Wout, bout):
    full = lambda shape: pl.BlockSpec(shape, lambda i: tuple(0 for _ in shape))
    return pl.pallas_call(
        _main_body,
        grid=(N // BM,),
        in_specs=[
            pl.BlockSpec((NC, BM, 16), lambda i: (0, i, 0)),
            pl.BlockSpec((NC, BM, 16), lambda i: (0, i, 0)),
            pl.BlockSpec((NC, BM, 16), lambda i: (0, i, 0)),
            pl.BlockSpec((NC, BM, 16), lambda i: (0, i, 0)),
            pl.BlockSpec((T, BM, 1), lambda i: (0, i, 0)),
            full((F, HID)), full((F, HID)), full((F, HID)),
            full((2 * HID, HID)), full((2 * HID, HID)), full((2 * HID, HID)),
            full((1, HID)), full((1, HID)), full((1, HID)),
            full((1, HID)), full((1, HID)), full((1, HID)),
            full((HID, OUT)), full((1, OUT)),
        ],
        out_specs=pl.BlockSpec((BM, OUT), lambda i: (i, 0)),
        out_shape=jax.ShapeDtypeStruct((N, OUT), _f32),
    )(*aggs, deg_r, Wz, Wr, Wh, Wlz, Wlr, Wlh,
      bz, br, bh, blz, blr, blh, Wout, bout)


def kernel(x, edge_index, Wz, bz, Wr, br, Wh, bh,
           Wlz, blz, Wlr, blr, Wlh, blh, Wout, bout):
    loop = jnp.broadcast_to(jnp.arange(N, dtype=jnp.int32), (T, 1, N))
    ei = jnp.concatenate([edge_index, jnp.tile(loop, (1, 2, 1))], axis=2)
    src = ei[:, 0, :]
    dst = ei[:, 1, :]
    pad = EP - E2
    src_p = jnp.pad(src, ((0, 0), (0, pad)), constant_values=N)
    dst_p = jnp.pad(dst, ((0, 0), (0, pad)), constant_values=N)
    src_p = src_p.reshape(T, NS, CPT, CHUNK)
    dst_p = dst_p.reshape(T, NS, CPT, CHUNK)
    x_s = jnp.stack([x[:, :, :16], x[:, :, 16:]])

    z1 = jnp.zeros((NROW,), _f32)
    z16 = jnp.zeros((NROW, 16), _f32)
    ones128 = jnp.ones((CHUNK,), _f32)

    deg = _deg_call(dst_p, z1, ones128)
    deg_r = deg.reshape(T, NROW, 1)
    aggs = []
    for t in range(T):
        tbl_t = _xn_call(x_s, deg_r, t)
        aggs.append(_agg_call(src_p, dst_p, tbl_t, z16, t))
    return _main_call(
        aggs, deg_r, Wz, Wr, Wh, Wlz, Wlr, Wlh,
        bz.reshape(1, HID), br.reshape(1, HID), bh.reshape(1, HID),
        blz.reshape(1, HID), blr.reshape(1, HID), blh.reshape(1, HID),
        Wout, bout.reshape(1, OUT))


# speedup vs baseline: 54.0559x; 1.1025x over previous
import functools

import jax
import jax.numpy as jnp
from jax import lax
from jax.experimental import pallas as pl
from jax.experimental.pallas import tpu as pltpu
from jax.experimental.pallas import tpu_sc as plsc

N = 100000
E = 1600000
T = 4
F = 32
HID = 64
OUT = 16

NC = 2
NS = 16
CHUNK = 128
KCH = 5
NBLK = 158
PAIRS = NBLK // 2
KCH_D = 5
NBLK_D = 158
CPT = NBLK * KCH
EPT = CPT * CHUNK
EP = NS * EPT
NROW = 100352
RPT = NROW // NS

BN2 = 3136
BM = 2000

_f32 = jnp.float32


def _deg_body(didx, z1, ones_h, deg_out, ones_v, idx_v, acc0, acc1, ssem):
    cid = lax.axis_index("c")
    sid = lax.axis_index("s")
    pltpu.sync_copy(ones_h, ones_v)
    for tt, acc in enumerate((acc0, acc1)):
        t = cid * (T // NC) + tt
        pltpu.sync_copy(z1.at[pl.ds(sid * RPT, RPT)],
                        acc.at[pl.ds(sid * RPT, RPT)])
        plsc.subcore_barrier()

        def body(b, carry, acc=acc, t=t):
            pltpu.sync_copy(didx.at[t, sid, pl.ds(b * KCH_D, KCH_D)], idx_v)
            descs = [
                pltpu.async_copy(ones_v, acc.at[idx_v.at[j]], ssem, add=True)
                for j in range(KCH_D)
            ]
            for d in descs:
                d.wait()
            return carry

        lax.fori_loop(0, NBLK_D, body, 0)
        plsc.subcore_barrier()
        pltpu.sync_copy(acc.at[pl.ds(sid * RPT, RPT)],
                        deg_out.at[t, pl.ds(sid * RPT, RPT)])
        plsc.subcore_barrier()


def _deg_call(dst_p, z1, ones128):
    mesh = plsc.VectorSubcoreMesh(core_axis_name="c", subcore_axis_name="s")
    kern = pl.kernel(
        _deg_body,
        out_type=jax.ShapeDtypeStruct((T, NROW), _f32),
        mesh=mesh,
        compiler_params=pltpu.CompilerParams(use_tc_tiling_on_sc=False),
        scratch_types=[
            pltpu.VMEM((CHUNK,), _f32),
            pltpu.VMEM((KCH_D, CHUNK), jnp.int32),
            pltpu.VMEM_SHARED((NROW,), _f32),
            pltpu.VMEM_SHARED((NROW,), _f32),
            pltpu.SemaphoreType.DMA,
        ],
    )
    return kern(dst_p, z1, ones128)


def _xn_body(x_ref, deg_ref, out_ref):
    i = pl.program_id(0)
    di = lax.rsqrt(deg_ref[0] + 1.0)
    xn = x_ref[0] * di
    glob = lax.broadcasted_iota(jnp.int32, (BN2, 1), 0) + i * BN2
    xn = jnp.where(glob < N, xn, 0.0)
    out_ref[0] = xn[:, :16]
    out_ref[1] = xn[:, 16:]


def _xn_call(x, deg_r, t):
    return pl.pallas_call(
        _xn_body,
        grid=(NROW // BN2,),
        in_specs=[
            pl.BlockSpec((1, BN2, F), lambda i: (t, i, 0)),
            pl.BlockSpec((1, BN2, 1), lambda i: (t, i, 0)),
        ],
        out_specs=pl.BlockSpec((NC, BN2, 16), lambda i: (0, i, 0)),
        out_shape=jax.ShapeDtypeStruct((NC, NROW, 16), _f32),
    )(x, deg_r)


def _agg_body(t, sidx, didx, tbl, agg_out,
              sidx_v, didx_v, rows, acc, gsems, ssems):
    cid = lax.axis_index("c")
    sid = lax.axis_index("s")
    if True:
        tbl_t = tbl.at[cid]
        pltpu.sync_copy(tbl_t.at[pl.ds(sid * RPT, RPT)],
                        acc.at[pl.ds(sid * RPT, RPT)])
        plsc.subcore_barrier()

        def fire_g(b, p, tbl_t=tbl_t, t=t):
            pltpu.sync_copy(sidx.at[t, sid, pl.ds(b * KCH, KCH)],
                            sidx_v.at[p])
            pltpu.sync_copy(didx.at[t, sid, pl.ds(b * KCH, KCH)],
                            didx_v.at[p])
            for j in range(KCH):
                pltpu.async_copy(tbl_t.at[sidx_v.at[p, j]], rows.at[p, j],
                                 gsems.at[p])

        def wait_g(p, tbl_t=tbl_t):
            for j in range(KCH):
                pltpu.make_async_copy(tbl_t.at[sidx_v.at[p, j]],
                                      rows.at[p, j], gsems.at[p]).wait()

        def fire_s(p):
            for j in range(KCH):
                pltpu.async_copy(rows.at[p, j], acc.at[didx_v.at[p, j]],
                                 ssems.at[p], add=True)

        def wait_s(p):
            for j in range(KCH):
                pltpu.make_async_copy(rows.at[p, j], acc.at[didx_v.at[p, j]],
                                      ssems.at[p]).wait()

        fire_g(0, 0)

        def body(i, carry):
            b = 2 * i
            wait_g(0)
            fire_g(b + 1, 1)
            fire_s(0)
            wait_g(1)
            wait_s(0)

            @pl.when(i < PAIRS - 1)
            def _():
                fire_g(b + 2, 0)
            fire_s(1)
            wait_s(1)
            return carry

        lax.fori_loop(0, PAIRS, body, 0)
        plsc.subcore_barrier()
        pltpu.sync_copy(acc.at[pl.ds(sid * RPT, RPT)],
                        agg_out.at[cid, pl.ds(sid * RPT, RPT)])
        plsc.subcore_barrier()


def _agg_call(src_p, dst_p, tbl, t):
    mesh = plsc.VectorSubcoreMesh(core_axis_name="c", subcore_axis_name="s")
    kern = pl.kernel(
        functools.partial(_agg_body, t),
        out_type=jax.ShapeDtypeStruct((NC, NROW, 16), _f32),
        mesh=mesh,
        compiler_params=pltpu.CompilerParams(use_tc_tiling_on_sc=False),
        scratch_types=[
            pltpu.VMEM((2, KCH, CHUNK), jnp.int32),
            pltpu.VMEM((2, KCH, CHUNK), jnp.int32),
            pltpu.VMEM((2, KCH, CHUNK, 16), _f32),
            pltpu.VMEM_SHARED((NROW, 16), _f32),
            pltpu.SemaphoreType.DMA((2,)),
            pltpu.SemaphoreType.DMA((2,)),
        ],
    )
    return kern(src_p, dst_p, tbl)


def _main_body(a0_ref, a1_ref, a2_ref, a3_ref, deg_ref,
               Wz_ref, Wr_ref, Wh_ref, Wlz_ref, Wlr_ref, Wlh_ref,
               bz_ref, br_ref, bh_ref, blz_ref, blr_ref, blh_ref,
               Wout_ref, bout_ref, out_ref):
    agg_refs = (a0_ref, a1_ref, a2_ref, a3_ref)
    dot = functools.partial(jnp.dot, preferred_element_type=_f32)
    Wlz = Wlz_ref[...]
    Wlr = Wlr_ref[...]
    Wlh = Wlh_ref[...]
    Az = dot(Wz_ref[...], Wlz[:HID])
    Ar = dot(Wr_ref[...], Wlr[:HID])
    Ah = dot(Wh_ref[...], Wlh[:HID])
    Bz, Br, Bh = Wlz[HID:], Wlr[HID:], Wlh[HID:]
    cz = dot(bz_ref[...], Wlz[:HID]) + blz_ref[...]
    cr = dot(br_ref[...], Wlr[:HID]) + blr_ref[...]
    ch = dot(bh_ref[...], Wlh[:HID]) + blh_ref[...]

    H = jnp.zeros((BM, HID), _f32)
    for t in range(T):
        di = lax.rsqrt(deg_ref[t] + 1.0)
        a0, a1 = agg_refs[t][0], agg_refs[t][1]
        xa0 = a0 * di
        xa1 = a1 * di
        Z = jax.nn.sigmoid(dot(xa0, Az[:16]) + dot(xa1, Az[16:])
                           + dot(H, Bz) + cz)
        R = jax.nn.sigmoid(dot(xa0, Ar[:16]) + dot(xa1, Ar[16:])
                           + dot(H, Br) + cr)
        Ht = jnp.tanh(dot(xa0, Ah[:16]) + dot(xa1, Ah[16:])
                      + dot(H * R, Bh) + ch)
        H = Z * H + (1.0 - Z) * Ht
    out_ref[...] = dot(H, Wout_ref[...]) + bout_ref[...]


def _main_call(aggs, deg_r, Wz, Wr, Wh, Wlz, Wlr, Wlh,
               bz, br, bh, blz, blr, blh, Wout, bout):
    full = lambda shape: pl.BlockSpec(shape, lambda i: tuple(0 for _ in shape))
    return pl.pallas_call(
        _main_body,
        grid=(N // BM,),
        in_specs=[
            pl.BlockSpec((NC, BM, 16), lambda i: (0, i, 0)),
            pl.BlockSpec((NC, BM, 16), lambda i: (0, i, 0)),
            pl.BlockSpec((NC, BM, 16), lambda i: (0, i, 0)),
            pl.BlockSpec((NC, BM, 16), lambda i: (0, i, 0)),
            pl.BlockSpec((T, BM, 1), lambda i: (0, i, 0)),
            full((F, HID)), full((F, HID)), full((F, HID)),
            full((2 * HID, HID)), full((2 * HID, HID)), full((2 * HID, HID)),
            full((1, HID)), full((1, HID)), full((1, HID)),
            full((1, HID)), full((1, HID)), full((1, HID)),
            full((HID, OUT)), full((1, OUT)),
        ],
        out_specs=pl.BlockSpec((BM, OUT), lambda i: (i, 0)),
        out_shape=jax.ShapeDtypeStruct((N, OUT), _f32),
    )(*aggs, deg_r, Wz, Wr, Wh, Wlz, Wlr, Wlh,
      bz, br, bh, blz, blr, blh, Wout, bout)


def kernel(x, edge_index, Wz, bz, Wr, br, Wh, bh,
           Wlz, blz, Wlr, blr, Wlh, blh, Wout, bout):
    src = edge_index[:, 0, :]
    dst = edge_index[:, 1, :]
    pad = EP - E
    src_p = jnp.pad(src, ((0, 0), (0, pad)), constant_values=N)
    dst_p = jnp.pad(dst, ((0, 0), (0, pad)), constant_values=N)
    src_p = src_p.reshape(T, NS, CPT, CHUNK)
    dst_p = dst_p.reshape(T, NS, CPT, CHUNK)

    z1 = jnp.zeros((NROW,), _f32)
    ones128 = jnp.ones((CHUNK,), _f32)

    deg = _deg_call(dst_p, z1, ones128)
    deg_r = deg.reshape(T, NROW, 1)
    aggs = []
    for t in range(T):
        tbl_t = _xn_call(x, deg_r, t)
        aggs.append(_agg_call(src_p, dst_p, tbl_t, t))
    return _main_call(
        aggs, deg_r, Wz, Wr, Wh, Wlz, Wlr, Wlh,
        bz.reshape(1, HID), br.reshape(1, HID), bh.reshape(1, HID),
        blz.reshape(1, HID), blr.reshape(1, HID), blh.reshape(1, HID),
        Wout, bout.reshape(1, OUT))


# speedup vs baseline: 54.8297x; 1.0143x over previous
import functools

import jax
import jax.numpy as jnp
from jax import lax
from jax.experimental import pallas as pl
from jax.experimental.pallas import tpu as pltpu
from jax.experimental.pallas import tpu_sc as plsc

N = 100000
E = 1600000
T = 4
F = 32
HID = 64
OUT = 16

NC = 2
NS = 16
CHUNK = 128
KCH = 6
NBLK = 132
PAIRS = NBLK // 2
KCH_D = 6
NBLK_D = 132
CPT = NBLK * KCH
EPT = CPT * CHUNK
EP = NS * EPT
NROW = 100352
RPT = NROW // NS

BN2 = 3136
BM = 2000

_f32 = jnp.float32


def _deg_body(didx, z1, ones_h, deg_out, ones_v, idx_v, acc0, acc1, ssem):
    cid = lax.axis_index("c")
    sid = lax.axis_index("s")
    pltpu.sync_copy(ones_h, ones_v)
    for tt, acc in enumerate((acc0, acc1)):
        t = cid * (T // NC) + tt
        pltpu.sync_copy(z1.at[pl.ds(sid * RPT, RPT)],
                        acc.at[pl.ds(sid * RPT, RPT)])
        plsc.subcore_barrier()

        def body(b, carry, acc=acc, t=t):
            pltpu.sync_copy(didx.at[t, sid, pl.ds(b * KCH_D, KCH_D)], idx_v)
            descs = [
                pltpu.async_copy(ones_v, acc.at[idx_v.at[j]], ssem, add=True)
                for j in range(KCH_D)
            ]
            for d in descs:
                d.wait()
            return carry

        lax.fori_loop(0, NBLK_D, body, 0)
        plsc.subcore_barrier()
        pltpu.sync_copy(acc.at[pl.ds(sid * RPT, RPT)],
                        deg_out.at[t, pl.ds(sid * RPT, RPT)])
        plsc.subcore_barrier()


def _deg_call(dst_p, z1, ones128):
    mesh = plsc.VectorSubcoreMesh(core_axis_name="c", subcore_axis_name="s")
    kern = pl.kernel(
        _deg_body,
        out_type=jax.ShapeDtypeStruct((T, NROW), _f32),
        mesh=mesh,
        compiler_params=pltpu.CompilerParams(use_tc_tiling_on_sc=False),
        scratch_types=[
            pltpu.VMEM((CHUNK,), _f32),
            pltpu.VMEM((KCH_D, CHUNK), jnp.int32),
            pltpu.VMEM_SHARED((NROW,), _f32),
            pltpu.VMEM_SHARED((NROW,), _f32),
            pltpu.SemaphoreType.DMA,
        ],
    )
    return kern(dst_p, z1, ones128)


def _xn_body(x_ref, deg_ref, out_ref):
    i = pl.program_id(0)
    di = lax.rsqrt(deg_ref[0] + 1.0)
    xn = x_ref[0] * di
    glob = lax.broadcasted_iota(jnp.int32, (BN2, 1), 0) + i * BN2
    xn = jnp.where(glob < N, xn, 0.0)
    out_ref[0] = xn[:, :16]
    out_ref[1] = xn[:, 16:]


def _xn_call(x, deg_r, t):
    return pl.pallas_call(
        _xn_body,
        grid=(NROW // BN2,),
        in_specs=[
            pl.BlockSpec((1, BN2, F), lambda i: (t, i, 0)),
            pl.BlockSpec((1, BN2, 1), lambda i: (t, i, 0)),
        ],
        out_specs=pl.BlockSpec((NC, BN2, 16), lambda i: (0, i, 0)),
        out_shape=jax.ShapeDtypeStruct((NC, NROW, 16), _f32),
    )(x, deg_r)


def _agg_body(t, sidx, didx, tbl, agg_out,
              sidx_v, didx_v, rows, acc, gsems, ssems):
    cid = lax.axis_index("c")
    sid = lax.axis_index("s")
    if True:
        tbl_t = tbl.at[cid]
        pltpu.sync_copy(tbl_t.at[pl.ds(sid * RPT, RPT)],
                        acc.at[pl.ds(sid * RPT, RPT)])
        plsc.subcore_barrier()

        def fire_g(b, p, tbl_t=tbl_t, t=t):
            pltpu.sync_copy(sidx.at[t, sid, pl.ds(b * KCH, KCH)],
                            sidx_v.at[p])
            pltpu.sync_copy(didx.at[t, sid, pl.ds(b * KCH, KCH)],
                            didx_v.at[p])
            for j in range(KCH):
                pltpu.async_copy(tbl_t.at[sidx_v.at[p, j]], rows.at[p, j],
                                 gsems.at[p])

        def wait_g(p, tbl_t=tbl_t):
            for j in range(KCH):
                pltpu.make_async_copy(tbl_t.at[sidx_v.at[p, j]],
                                      rows.at[p, j], gsems.at[p]).wait()

        def fire_s(p):
            for j in range(KCH):
                pltpu.async_copy(rows.at[p, j], acc.at[didx_v.at[p, j]],
                                 ssems.at[p], add=True)

        def wait_s(p):
            for j in range(KCH):
                pltpu.make_async_copy(rows.at[p, j], acc.at[didx_v.at[p, j]],
                                      ssems.at[p]).wait()

        fire_g(0, 0)

        def body(i, carry):
            b = 2 * i
            wait_g(0)
            fire_g(b + 1, 1)
            fire_s(0)
            wait_g(1)
            wait_s(0)

            @pl.when(i < PAIRS - 1)
            def _():
                fire_g(b + 2, 0)
            fire_s(1)
            wait_s(1)
            return carry

        lax.fori_loop(0, PAIRS, body, 0)
        plsc.subcore_barrier()
        pltpu.sync_copy(acc.at[pl.ds(sid * RPT, RPT)],
                        agg_out.at[cid, pl.ds(sid * RPT, RPT)])
        plsc.subcore_barrier()


def _agg_call(src_p, dst_p, tbl, t):
    mesh = plsc.VectorSubcoreMesh(core_axis_name="c", subcore_axis_name="s")
    kern = pl.kernel(
        functools.partial(_agg_body, t),
        out_type=jax.ShapeDtypeStruct((NC, NROW, 16), _f32),
        mesh=mesh,
        compiler_params=pltpu.CompilerParams(use_tc_tiling_on_sc=False),
        scratch_types=[
            pltpu.VMEM((2, KCH, CHUNK), jnp.int32),
            pltpu.VMEM((2, KCH, CHUNK), jnp.int32),
            pltpu.VMEM((2, KCH, CHUNK, 16), _f32),
            pltpu.VMEM_SHARED((NROW, 16), _f32),
            pltpu.SemaphoreType.DMA((2,)),
            pltpu.SemaphoreType.DMA((2,)),
        ],
    )
    return kern(src_p, dst_p, tbl)


def _main_body(a0_ref, a1_ref, a2_ref, a3_ref, deg_ref,
               Wz_ref, Wr_ref, Wh_ref, Wlz_ref, Wlr_ref, Wlh_ref,
               bz_ref, br_ref, bh_ref, blz_ref, blr_ref, blh_ref,
               Wout_ref, bout_ref, out_ref):
    agg_refs = (a0_ref, a1_ref, a2_ref, a3_ref)
    dot = functools.partial(jnp.dot, preferred_element_type=_f32)
    Wlz = Wlz_ref[...]
    Wlr = Wlr_ref[...]
    Wlh = Wlh_ref[...]
    Az = dot(Wz_ref[...], Wlz[:HID])
    Ar = dot(Wr_ref[...], Wlr[:HID])
    Ah = dot(Wh_ref[...], Wlh[:HID])
    Bz, Br, Bh = Wlz[HID:], Wlr[HID:], Wlh[HID:]
    cz = dot(bz_ref[...], Wlz[:HID]) + blz_ref[...]
    cr = dot(br_ref[...], Wlr[:HID]) + blr_ref[...]
    ch = dot(bh_ref[...], Wlh[:HID]) + blh_ref[...]

    H = jnp.zeros((BM, HID), _f32)
    for t in range(T):
        di = lax.rsqrt(deg_ref[t] + 1.0)
        a0, a1 = agg_refs[t][0], agg_refs[t][1]
        xa0 = a0 * di
        xa1 = a1 * di
        Z = jax.nn.sigmoid(dot(xa0, Az[:16]) + dot(xa1, Az[16:])
                           + dot(H, Bz) + cz)
        R = jax.nn.sigmoid(dot(xa0, Ar[:16]) + dot(xa1, Ar[16:])
                           + dot(H, Br) + cr)
        Ht = jnp.tanh(dot(xa0, Ah[:16]) + dot(xa1, Ah[16:])
                      + dot(H * R, Bh) + ch)
        H = Z * H + (1.0 - Z) * Ht
    out_ref[...] = dot(H, Wout_ref[...]) + bout_ref[...]


def _main_call(aggs, deg_r, Wz, Wr, Wh, Wlz, Wlr, Wlh,
               bz, br, bh, blz, blr, blh, Wout, bout):
    full = lambda shape: pl.BlockSpec(shape, lambda i: tuple(0 for _ in shape))
    return pl.pallas_call(
        _main_body,
        grid=(N // BM,),
        in_specs=[
            pl.BlockSpec((NC, BM, 16), lambda i: (0, i, 0)),
            pl.BlockSpec((NC, BM, 16), lambda i: (0, i, 0)),
            pl.BlockSpec((NC, BM, 16), lambda i: (0, i, 0)),
            pl.BlockSpec((NC, BM, 16), lambda i: (0, i, 0)),
            pl.BlockSpec((T, BM, 1), lambda i: (0, i, 0)),
            full((F, HID)), full((F, HID)), full((F, HID)),
            full((2 * HID, HID)), full((2 * HID, HID)), full((2 * HID, HID)),
            full((1, HID)), full((1, HID)), full((1, HID)),
            full((1, HID)), full((1, HID)), full((1, HID)),
            full((HID, OUT)), full((1, OUT)),
        ],
        out_specs=pl.BlockSpec((BM, OUT), lambda i: (i, 0)),
        out_shape=jax.ShapeDtypeStruct((N, OUT), _f32),
    )(*aggs, deg_r, Wz, Wr, Wh, Wlz, Wlr, Wlh,
      bz, br, bh, blz, blr, blh, Wout, bout)


def kernel(x, edge_index, Wz, bz, Wr, br, Wh, bh,
           Wlz, blz, Wlr, blr, Wlh, blh, Wout, bout):
    src = edge_index[:, 0, :]
    dst = edge_index[:, 1, :]
    pad = EP - E
    src_p = jnp.pad(src, ((0, 0), (0, pad)), constant_values=N)
    dst_p = jnp.pad(dst, ((0, 0), (0, pad)), constant_values=N)
    src_p = src_p.reshape(T, NS, CPT, CHUNK)
    dst_p = dst_p.reshape(T, NS, CPT, CHUNK)

    z1 = jnp.zeros((NROW,), _f32)
    ones128 = jnp.ones((CHUNK,), _f32)

    deg = _deg_call(dst_p, z1, ones128)
    deg_r = deg.reshape(T, NROW, 1)
    aggs = []
    for t in range(T):
        tbl_t = _xn_call(x, deg_r, t)
        aggs.append(_agg_call(src_p, dst_p, tbl_t, t))
    return _main_call(
        aggs, deg_r, Wz, Wr, Wh, Wlz, Wlr, Wlh,
        bz.reshape(1, HID), br.reshape(1, HID), bh.reshape(1, HID),
        blz.reshape(1, HID), blr.reshape(1, HID), blh.reshape(1, HID),
        Wout, bout.reshape(1, OUT))
